# Initial kernel scaffold; baseline (speedup 1.0000x reference)
import jax
import jax.numpy as jnp
from jax.experimental import pallas as pl


def kernel(x, W_enc_high, W_enc_mid, W_enc_low, cb_high, cb_mid, cb_low, W_dec):
    raise NotImplementedError("write your pallas kernel here")


# speedup vs baseline: 1.4661x; 1.4661x over previous
import jax
import jax.numpy as jnp
from jax.experimental import pallas as pl

B, C, H, W = 8, 3, 512, 512
D = 64
P = 8
G = 64
F = C * P * P


def _vq(zflat, cb):
    rown = jnp.sum(zflat * zflat, axis=-1, keepdims=True)
    cbn = jnp.sum(cb * cb, axis=-1)
    scores = jax.lax.dot_general(zflat, cb, (((1,), (1,)), ((), ())),
                                 preferred_element_type=jnp.float32)
    d = rown - 2.0 * scores + cbn[None, :]
    idx = jnp.argmin(d, axis=-1).astype(jnp.int32)
    onehot = (jax.lax.broadcasted_iota(jnp.int32, d.shape, 1)
              == idx[:, None]).astype(jnp.float32)
    zq = jnp.dot(onehot, cb, preferred_element_type=jnp.float32)
    return zq, idx


def _fused(p8_ref, wh_ref, wm_ref, wl_ref, cbh_ref, cbm_ref, cbl_ref, wd_ref,
           out_ref, zh_ref, zm_ref, zl_ref, qh_ref, qm_ref, ql_ref,
           ih_ref, im_ref, il_ref):
    t = p8_ref[0]
    tf = t.reshape(G * G, F)
    z_h = jnp.dot(tf, wh_ref[...], preferred_element_type=jnp.float32)

    t4 = t.reshape(32, 2, 32, 2, F)
    z_m = jnp.zeros((32 * 32, D), jnp.float32)
    for di in range(2):
        for dj in range(2):
            sub = t4[:, di, :, dj, :].reshape(32 * 32, F)
            z_m = z_m + jnp.dot(sub, wm_ref[2 * di + dj],
                                preferred_element_type=jnp.float32)

    t16 = t.reshape(16, 4, 16, 4, F)
    z_l = jnp.zeros((16 * 16, D), jnp.float32)
    for di in range(4):
        for dj in range(4):
            sub = t16[:, di, :, dj, :].reshape(16 * 16, F)
            z_l = z_l + jnp.dot(sub, wl_ref[4 * di + dj],
                                preferred_element_type=jnp.float32)

    q_h, i_h = _vq(z_h, cbh_ref[...])
    q_m, i_m = _vq(z_m, cbm_ref[...])
    q_l, i_l = _vq(z_l, cbl_ref[...])

    zh_ref[0] = z_h.reshape(G, G, D)
    zm_ref[0] = z_m.reshape(32, 32, D)
    zl_ref[0] = z_l.reshape(16, 16, D)
    qh_ref[0] = q_h.reshape(G, G, D)
    qm_ref[0] = q_m.reshape(32, 32, D)
    ql_ref[0] = q_l.reshape(16, 16, D)
    ih_ref[0] = i_h.reshape(G, G)
    im_ref[0] = i_m.reshape(32, 32)
    il_ref[0] = i_l.reshape(16, 16)

    d_h = z_h + (q_h - z_h)
    d_m = z_m + (q_m - z_m)
    d_l = z_l + (q_l - z_l)
    up_m = jnp.broadcast_to(d_m.reshape(32, 1, 32, 1, D),
                            (32, 2, 32, 2, D)).reshape(G * G, D)
    up_l = jnp.broadcast_to(d_l.reshape(16, 1, 16, 1, D),
                            (16, 4, 16, 4, D)).reshape(G * G, D)
    h = d_h + up_m + up_l
    out = jnp.dot(h, wd_ref[...], preferred_element_type=jnp.float32)
    out_ref[0] = out.reshape(G, G, F)


def kernel(x, W_enc_high, W_enc_mid, W_enc_low, cb_high, cb_mid, cb_low, W_dec):
    p8 = x.reshape(B, C, G, P, G, P).transpose(0, 2, 4, 1, 3, 5).reshape(B, G, G, F)
    wm = W_enc_mid.reshape(C, 2, P, 2, P, D).transpose(1, 3, 0, 2, 4, 5).reshape(4, F, D)
    wl = W_enc_low.reshape(C, 4, P, 4, P, D).transpose(1, 3, 0, 2, 4, 5).reshape(16, F, D)

    full = lambda shape: pl.BlockSpec(shape, lambda b: (0,) * len(shape))
    outs = pl.pallas_call(
        _fused,
        grid=(B,),
        in_specs=[
            pl.BlockSpec((1, G, G, F), lambda b: (b, 0, 0, 0)),
            full((F, D)),
            full((4, F, D)),
            full((16, F, D)),
            full((256, D)),
            full((128, D)),
            full((128, D)),
            full((D, F)),
        ],
        out_specs=[
            pl.BlockSpec((1, G, G, F), lambda b: (b, 0, 0, 0)),
            pl.BlockSpec((1, G, G, D), lambda b: (b, 0, 0, 0)),
            pl.BlockSpec((1, 32, 32, D), lambda b: (b, 0, 0, 0)),
            pl.BlockSpec((1, 16, 16, D), lambda b: (b, 0, 0, 0)),
            pl.BlockSpec((1, G, G, D), lambda b: (b, 0, 0, 0)),
            pl.BlockSpec((1, 32, 32, D), lambda b: (b, 0, 0, 0)),
            pl.BlockSpec((1, 16, 16, D), lambda b: (b, 0, 0, 0)),
            pl.BlockSpec((1, G, G), lambda b: (b, 0, 0)),
            pl.BlockSpec((1, 32, 32), lambda b: (b, 0, 0)),
            pl.BlockSpec((1, 16, 16), lambda b: (b, 0, 0)),
        ],
        out_shape=[
            jax.ShapeDtypeStruct((B, G, G, F), jnp.float32),
            jax.ShapeDtypeStruct((B, G, G, D), jnp.float32),
            jax.ShapeDtypeStruct((B, 32, 32, D), jnp.float32),
            jax.ShapeDtypeStruct((B, 16, 16, D), jnp.float32),
            jax.ShapeDtypeStruct((B, G, G, D), jnp.float32),
            jax.ShapeDtypeStruct((B, 32, 32, D), jnp.float32),
            jax.ShapeDtypeStruct((B, 16, 16, D), jnp.float32),
            jax.ShapeDtypeStruct((B, G, G), jnp.int32),
            jax.ShapeDtypeStruct((B, 32, 32), jnp.int32),
            jax.ShapeDtypeStruct((B, 16, 16), jnp.int32),
        ],
    )(p8, W_enc_high, wm, wl, cb_high, cb_mid, cb_low, W_dec)

    out_pre, z_h, z_m, z_l, q_h, q_m, q_l, i_h, i_m, i_l = outs
    x_rec = out_pre.reshape(B, G, G, C, P, P).transpose(0, 3, 1, 4, 2, 5).reshape(B, C, H, W)
    return (x_rec, (z_h, z_m, z_l), (q_h, q_m, q_l), (i_h, i_m, i_l))
